# Initial kernel scaffold; baseline (speedup 1.0000x reference)
import jax
import jax.numpy as jnp
from jax.experimental import pallas as pl


def kernel(x, edge_index, W1, b1, W2, b2, W3, b3, Wc, bc):
    raise NotImplementedError("write your pallas kernel here")


# speedup vs baseline: 75.8297x; 75.8297x over previous
import functools

import jax
import jax.numpy as jnp
from jax import lax
from jax.experimental import pallas as pl
from jax.experimental.pallas import tpu as pltpu
from jax.experimental.pallas import tpu_sc as plsc

N = 10000
E = 320000
NC = 2
NS = 16
NW = NC * NS
EPW = E // NW
LANES = 16
CHUNKS = EPW // LANES

_SC_MESH = dict(core_axis_name="c", subcore_axis_name="s",
                num_cores=NC, num_subcores=NS)


def _wid():
    return lax.axis_index("s") * NC + lax.axis_index("c")


def _deg_body(dst_hbm, out_hbm, dst_v, acc_v):
    w = _wid()
    pltpu.sync_copy(dst_hbm.at[w], dst_v)
    one = jnp.ones((LANES,), jnp.float32)
    zero = jnp.zeros((LANES,), jnp.float32)

    def zero_it(i, _):
        acc_v[pl.ds(i * LANES, LANES)] = zero
        return 0

    lax.fori_loop(0, N // LANES, zero_it, 0)

    def body(i, _):
        d = dst_v[pl.ds(i * LANES, LANES)]
        plsc.addupdate_scatter(acc_v, [d], one)
        return 0

    lax.fori_loop(0, CHUNKS, body, 0)
    pltpu.sync_copy(acc_v, out_hbm.at[w])


_SC_PARAMS = pltpu.CompilerParams(needs_layout_passes=False)

_deg_kernel = functools.partial(
    pl.kernel,
    out_type=jax.ShapeDtypeStruct((NW, N), jnp.float32),
    mesh=plsc.VectorSubcoreMesh(**_SC_MESH),
    compiler_params=_SC_PARAMS,
    scratch_types=[
        pltpu.VMEM((EPW,), jnp.int32),
        pltpu.VMEM((N,), jnp.float32),
    ],
)(_deg_body)


def _agg_body(k_planes, compute_norm, *refs):
    if compute_norm:
        (hw_hbm, src_hbm, dst_hbm, dinv_hbm, out_hbm, norm_out,
         src_v, dst_v, norm_v, dinv_v, hw_v, acc_v) = refs
    else:
        (hw_hbm, src_hbm, dst_hbm, norm_hbm, out_hbm,
         src_v, dst_v, norm_v, hw_v, acc_v) = refs
    w = _wid()
    pltpu.sync_copy(src_hbm.at[w], src_v)
    pltpu.sync_copy(dst_hbm.at[w], dst_v)
    if compute_norm:
        pltpu.sync_copy(dinv_hbm.at[0], dinv_v)
    else:
        pltpu.sync_copy(norm_hbm.at[w], norm_v)
    pltpu.sync_copy(hw_hbm, hw_v)

    zero = jnp.zeros((LANES,), jnp.float32)

    def zero_it(i, _):
        for k in range(k_planes):
            acc_v[k, pl.ds(i * LANES, LANES)] = zero
        return 0

    lax.fori_loop(0, N // LANES, zero_it, 0)

    def body(i, _):
        sl = pl.ds(i * LANES, LANES)
        vs = src_v[sl]
        vd = dst_v[sl]
        if compute_norm:
            vn = plsc.load_gather(dinv_v, [vs]) * plsc.load_gather(dinv_v, [vd])
            norm_v[sl] = vn
        else:
            vn = norm_v[sl]
        for k in range(k_planes):
            kk = jnp.full((LANES,), k, jnp.int32)
            g = plsc.load_gather(hw_v, [kk, vs])
            plsc.addupdate_scatter(acc_v, [kk, vd], g * vn)
        return 0

    lax.fori_loop(0, CHUNKS, body, 0)
    pltpu.sync_copy(acc_v, out_hbm.at[w])
    if compute_norm:
        pltpu.sync_copy(norm_v, norm_out.at[w])


def _make_agg_kernel(k_planes, compute_norm):
    outs = [jax.ShapeDtypeStruct((NW, k_planes, N), jnp.float32)]
    scratch = [
        pltpu.VMEM((EPW,), jnp.int32),
        pltpu.VMEM((EPW,), jnp.int32),
        pltpu.VMEM((EPW,), jnp.float32),
    ]
    if compute_norm:
        outs.append(jax.ShapeDtypeStruct((NW, EPW), jnp.float32))
        scratch.append(pltpu.VMEM((N,), jnp.float32))
    scratch += [
        pltpu.VMEM((k_planes, N), jnp.float32),
        pltpu.VMEM((k_planes, N), jnp.float32),
    ]
    return functools.partial(
        pl.kernel,
        out_type=tuple(outs) if len(outs) > 1 else outs[0],
        mesh=plsc.VectorSubcoreMesh(**_SC_MESH),
        compiler_params=_SC_PARAMS,
        scratch_types=scratch,
    )(functools.partial(_agg_body, k_planes, compute_norm))


_agg1 = _make_agg_kernel(4, True)
_agg2 = _make_agg_kernel(4, False)
_agg3 = _make_agg_kernel(2, False)


def _prep_body(dp_ref, x_ref, w1_ref, dinv_ref, dinv2_ref, hw1_ref):
    deg = jnp.ones((1, N), jnp.float32)
    for i in range(NW):
        deg = deg + dp_ref[i][None, :]
    dinv_ref[...] = lax.rsqrt(deg)
    dinv2_ref[...] = 1.0 / deg
    hw1_ref[...] = jnp.dot(x_ref[...], w1_ref[...],
                           preferred_element_type=jnp.float32)


def _tc_prep(deg_partials, x, W1):
    return pl.pallas_call(
        _prep_body,
        out_shape=(
            jax.ShapeDtypeStruct((1, N), jnp.float32),
            jax.ShapeDtypeStruct((1, N), jnp.float32),
            jax.ShapeDtypeStruct((N, 4), jnp.float32),
        ),
    )(deg_partials, x, W1)


def _dense_body(k_planes, next_w, p_ref, hw_ref, dinv2_ref, b_ref, wT_ref,
                bo_ref, h_ref, hwn_ref):
    agg = p_ref[0]
    for i in range(1, NW):
        agg = agg + p_ref[i]
    agg = agg + hw_ref[...] * dinv2_ref[...] + b_ref[...]
    h = jnp.tanh(agg)
    h_ref[...] = h
    hwn_ref[...] = jnp.dot(wT_ref[...], h,
                           preferred_element_type=jnp.float32) + bo_ref[...]


def _tc_dense(k_planes, next_w, partials, hw, dinv2, b_col, WT, bo_col):
    return pl.pallas_call(
        functools.partial(_dense_body, k_planes, next_w),
        out_shape=(
            jax.ShapeDtypeStruct((k_planes, N), jnp.float32),
            jax.ShapeDtypeStruct((next_w, N), jnp.float32),
        ),
    )(partials, hw, dinv2, b_col, WT, bo_col)


def kernel(x, edge_index, W1, b1, W2, b2, W3, b3, Wc, bc):
    src = edge_index[0].reshape(NW, EPW)
    dst = edge_index[1].reshape(NW, EPW)

    deg_partials = _deg_kernel(dst)
    dinv, dinv2, hw1_rows = _tc_prep(deg_partials, x, W1)
    hw1 = hw1_rows.T

    z4 = jnp.zeros((4, 1), jnp.float32)
    z2 = jnp.zeros((2, 1), jnp.float32)

    p1, norm = _agg1(hw1, src, dst, dinv)
    h1, hw2 = _tc_dense(4, 4, p1, hw1, dinv2, b1.reshape(4, 1), W2.T, z4)

    p2 = _agg2(hw2, src, dst, norm)
    h2, hw3 = _tc_dense(4, 2, p2, hw2, dinv2, b2.reshape(4, 1), W3.T, z2)

    p3 = _agg3(hw3, src, dst, norm)
    h3, outp = _tc_dense(2, 10, p3, hw3, dinv2, b3.reshape(2, 1), Wc.T,
                         bc.reshape(10, 1))

    return (outp.T, h3.T)


# speedup vs baseline: 105.7843x; 1.3950x over previous
import functools

import jax
import jax.numpy as jnp
from jax import lax
from jax.experimental import pallas as pl
from jax.experimental.pallas import tpu as pltpu
from jax.experimental.pallas import tpu_sc as plsc

N = 10000
E = 320000
NC = 2
NS = 16
NW = NC * NS
EPW = E // NW
LANES = 16
CHUNKS = EPW // LANES

_SC_MESH = dict(core_axis_name="c", subcore_axis_name="s",
                num_cores=NC, num_subcores=NS)


def _wid():
    return lax.axis_index("s") * NC + lax.axis_index("c")


def _deg_body(dst_hbm, out_hbm, dst_v, acc_v):
    w = _wid()
    pltpu.sync_copy(dst_hbm.at[w], dst_v)
    one = jnp.ones((LANES,), jnp.float32)
    zero = jnp.zeros((LANES,), jnp.float32)

    @plsc.parallel_loop(0, N // LANES, unroll=8)
    def _(i):
        acc_v[pl.ds(i * LANES, LANES)] = zero

    @plsc.parallel_loop(0, CHUNKS, unroll=8)
    def _(i):
        d = dst_v[pl.ds(i * LANES, LANES)]
        plsc.addupdate_scatter(acc_v, [d], one)

    pltpu.sync_copy(acc_v, out_hbm.at[w])


_SC_PARAMS = pltpu.CompilerParams(needs_layout_passes=False)

_deg_kernel = functools.partial(
    pl.kernel,
    out_type=jax.ShapeDtypeStruct((NW, N), jnp.float32),
    mesh=plsc.VectorSubcoreMesh(**_SC_MESH),
    compiler_params=_SC_PARAMS,
    scratch_types=[
        pltpu.VMEM((EPW,), jnp.int32),
        pltpu.VMEM((N,), jnp.float32),
    ],
)(_deg_body)


def _agg_body(k_planes, compute_norm, *refs):
    if compute_norm:
        (hw_hbm, src_hbm, dst_hbm, dinv_hbm, out_hbm, norm_out,
         src_v, dst_v, norm_v, dinv_v, hw_v, acc_v) = refs
    else:
        (hw_hbm, src_hbm, dst_hbm, norm_hbm, out_hbm,
         src_v, dst_v, norm_v, hw_v, acc_v) = refs
    w = _wid()
    pltpu.sync_copy(src_hbm.at[w], src_v)
    pltpu.sync_copy(dst_hbm.at[w], dst_v)
    if compute_norm:
        pltpu.sync_copy(dinv_hbm.at[0], dinv_v)
    else:
        pltpu.sync_copy(norm_hbm.at[w], norm_v)
    pltpu.sync_copy(hw_hbm, hw_v)

    zero = jnp.zeros((LANES,), jnp.float32)

    @plsc.parallel_loop(0, N // LANES, unroll=8)
    def _(i):
        for k in range(k_planes):
            acc_v[k, pl.ds(i * LANES, LANES)] = zero

    @plsc.parallel_loop(0, CHUNKS, unroll=4)
    def _(i):
        sl = pl.ds(i * LANES, LANES)
        vs = src_v[sl]
        vd = dst_v[sl]
        if compute_norm:
            vn = plsc.load_gather(dinv_v, [vs]) * plsc.load_gather(dinv_v, [vd])
            norm_v[sl] = vn
        else:
            vn = norm_v[sl]
        for k in range(k_planes):
            kk = jnp.full((LANES,), k, jnp.int32)
            g = plsc.load_gather(hw_v, [kk, vs])
            plsc.addupdate_scatter(acc_v, [kk, vd], g * vn)
    pltpu.sync_copy(acc_v, out_hbm.at[w])
    if compute_norm:
        pltpu.sync_copy(norm_v, norm_out.at[w])


def _make_agg_kernel(k_planes, compute_norm):
    outs = [jax.ShapeDtypeStruct((NW, k_planes, N), jnp.float32)]
    scratch = [
        pltpu.VMEM((EPW,), jnp.int32),
        pltpu.VMEM((EPW,), jnp.int32),
        pltpu.VMEM((EPW,), jnp.float32),
    ]
    if compute_norm:
        outs.append(jax.ShapeDtypeStruct((NW, EPW), jnp.float32))
        scratch.append(pltpu.VMEM((N,), jnp.float32))
    scratch += [
        pltpu.VMEM((k_planes, N), jnp.float32),
        pltpu.VMEM((k_planes, N), jnp.float32),
    ]
    return functools.partial(
        pl.kernel,
        out_type=tuple(outs) if len(outs) > 1 else outs[0],
        mesh=plsc.VectorSubcoreMesh(**_SC_MESH),
        compiler_params=_SC_PARAMS,
        scratch_types=scratch,
    )(functools.partial(_agg_body, k_planes, compute_norm))


_agg1 = _make_agg_kernel(4, True)
_agg2 = _make_agg_kernel(4, False)
_agg3 = _make_agg_kernel(2, False)


def _prep_body(dp_ref, x_ref, w1_ref, dinv_ref, dinv2_ref, hw1_ref):
    deg = jnp.ones((1, N), jnp.float32)
    for i in range(NW):
        deg = deg + dp_ref[i][None, :]
    dinv_ref[...] = lax.rsqrt(deg)
    dinv2_ref[...] = 1.0 / deg
    hw1_ref[...] = jnp.dot(x_ref[...], w1_ref[...],
                           preferred_element_type=jnp.float32)


def _tc_prep(deg_partials, x, W1):
    return pl.pallas_call(
        _prep_body,
        out_shape=(
            jax.ShapeDtypeStruct((1, N), jnp.float32),
            jax.ShapeDtypeStruct((1, N), jnp.float32),
            jax.ShapeDtypeStruct((N, 4), jnp.float32),
        ),
    )(deg_partials, x, W1)


def _dense_body(k_planes, next_w, p_ref, hw_ref, dinv2_ref, b_ref, wT_ref,
                bo_ref, h_ref, hwn_ref):
    agg = p_ref[0]
    for i in range(1, NW):
        agg = agg + p_ref[i]
    agg = agg + hw_ref[...] * dinv2_ref[...] + b_ref[...]
    h = jnp.tanh(agg)
    h_ref[...] = h
    hwn_ref[...] = jnp.dot(wT_ref[...], h,
                           preferred_element_type=jnp.float32) + bo_ref[...]


def _tc_dense(k_planes, next_w, partials, hw, dinv2, b_col, WT, bo_col):
    return pl.pallas_call(
        functools.partial(_dense_body, k_planes, next_w),
        out_shape=(
            jax.ShapeDtypeStruct((k_planes, N), jnp.float32),
            jax.ShapeDtypeStruct((next_w, N), jnp.float32),
        ),
    )(partials, hw, dinv2, b_col, WT, bo_col)


def kernel(x, edge_index, W1, b1, W2, b2, W3, b3, Wc, bc):
    src = edge_index[0].reshape(NW, EPW)
    dst = edge_index[1].reshape(NW, EPW)

    deg_partials = _deg_kernel(dst)
    dinv, dinv2, hw1_rows = _tc_prep(deg_partials, x, W1)
    hw1 = hw1_rows.T

    z4 = jnp.zeros((4, 1), jnp.float32)
    z2 = jnp.zeros((2, 1), jnp.float32)

    p1, norm = _agg1(hw1, src, dst, dinv)
    h1, hw2 = _tc_dense(4, 4, p1, hw1, dinv2, b1.reshape(4, 1), W2.T, z4)

    p2 = _agg2(hw2, src, dst, norm)
    h2, hw3 = _tc_dense(4, 2, p2, hw2, dinv2, b2.reshape(4, 1), W3.T, z2)

    p3 = _agg3(hw3, src, dst, norm)
    h3, outp = _tc_dense(2, 10, p3, hw3, dinv2, b3.reshape(2, 1), Wc.T,
                         bc.reshape(10, 1))

    return (outp.T, h3.T)
